# Initial kernel scaffold; baseline (speedup 1.0000x reference)
import jax
import jax.numpy as jnp
from jax.experimental import pallas as pl


def kernel(user, quiz, time, user_table, quiz_table, W1, b1, W2, b2):
    raise NotImplementedError("write your pallas kernel here")


# speedup vs baseline: 1.1536x; 1.1536x over previous
import functools

import jax
import jax.numpy as jnp
from jax import lax
from jax.experimental import pallas as pl
from jax.experimental.pallas import tpu as pltpu
from jax.experimental.pallas import tpu_sc as plsc

_BATCH = 16384
_EMB = 64
_HID = 32

_NC = 2
_NS = 16
_NW = _NC * _NS
_BPW = _BATCH // _NW


@functools.cache
def _make_sc_gather():
    @functools.partial(
        pl.kernel,
        mesh=plsc.VectorSubcoreMesh(
            core_axis_name="c", subcore_axis_name="s",
            num_cores=_NC, num_subcores=_NS,
        ),
        out_type=[
            jax.ShapeDtypeStruct((_BATCH, _EMB), jnp.float32),
            jax.ShapeDtypeStruct((_BATCH, _EMB), jnp.float32),
        ],
        scratch_types=[
            pltpu.VMEM((_BPW,), jnp.int32),
            pltpu.VMEM((_BPW, _EMB), jnp.float32),
            pltpu.VMEM((_BPW,), jnp.int32),
            pltpu.VMEM((_BPW, _EMB), jnp.float32),
            pltpu.SemaphoreType.DMA,
            pltpu.SemaphoreType.DMA,
        ],
        compiler_params=pltpu.CompilerParams(use_tc_tiling_on_sc=False),
    )
    def sc_gather(user_hbm, quiz_hbm, ut_hbm, qt_hbm, u_out, q_out,
                  uidx_v, urows_v, qidx_v, qrows_v, sem_u, sem_q):
        wid = lax.axis_index("s") * _NC + lax.axis_index("c")
        base = wid * _BPW
        pltpu.sync_copy(user_hbm.at[pl.ds(base, _BPW)], uidx_v)
        pltpu.sync_copy(quiz_hbm.at[pl.ds(base, _BPW)], qidx_v)
        cu = pltpu.async_copy(ut_hbm.at[uidx_v], urows_v, sem_u)
        cq = pltpu.async_copy(qt_hbm.at[qidx_v], qrows_v, sem_q)
        cu.wait()
        pltpu.sync_copy(urows_v, u_out.at[pl.ds(base, _BPW)])
        cq.wait()
        pltpu.sync_copy(qrows_v, q_out.at[pl.ds(base, _BPW)])

    return sc_gather


def _mlp_body(u_ref, q_ref, t_ref, w1u_ref, w1q_ref, w1t_ref, b1_ref,
              w2_ref, b2_ref, out_ref):
    h = (
        jnp.dot(u_ref[...], w1u_ref[...], preferred_element_type=jnp.float32)
        + jnp.dot(q_ref[...], w1q_ref[...], preferred_element_type=jnp.float32)
        + t_ref[...] * w1t_ref[...]
        + b1_ref[...]
    )
    h = jnp.maximum(h, 0.0)
    o = jnp.dot(h, w2_ref[...], preferred_element_type=jnp.float32) + b2_ref[...]
    out_ref[...] = 1.0 / (1.0 + jnp.exp(-o))


_MLP_BLK = 2048


def _mlp(u, q, time, W1u, W1q, W1t, b1, W2, b2):
    grid = (_BATCH // _MLP_BLK,)
    full = lambda shape: pl.BlockSpec(shape, lambda i: (0, 0))
    return pl.pallas_call(
        _mlp_body,
        grid=grid,
        in_specs=[
            pl.BlockSpec((_MLP_BLK, _EMB), lambda i: (i, 0)),
            pl.BlockSpec((_MLP_BLK, _EMB), lambda i: (i, 0)),
            pl.BlockSpec((_MLP_BLK, 1), lambda i: (i, 0)),
            full((_EMB, _HID)),
            full((_EMB, _HID)),
            full((1, _HID)),
            full((1, _HID)),
            full((_HID, 1)),
            full((1, 1)),
        ],
        out_specs=pl.BlockSpec((_MLP_BLK, 1), lambda i: (i, 0)),
        out_shape=jax.ShapeDtypeStruct((_BATCH, 1), jnp.float32),
    )(u, q, time, W1u, W1q, W1t, b1, W2, b2)


def kernel(user, quiz, time, user_table, quiz_table, W1, b1, W2, b2):
    u, q = _make_sc_gather()(user, quiz, user_table, quiz_table)
    W1u = W1[:_EMB]
    W1q = W1[_EMB:2 * _EMB]
    W1t = W1[2 * _EMB:]
    out = _mlp(u, q, time, W1u, W1q, W1t, b1.reshape(1, _HID), W2,
               b2.reshape(1, 1))
    return out[:, 0]


# speedup vs baseline: 1.2786x; 1.1084x over previous
import functools

import jax
import jax.numpy as jnp
from jax import lax
from jax.experimental import pallas as pl
from jax.experimental.pallas import tpu as pltpu
from jax.experimental.pallas import tpu_sc as plsc

_BATCH = 16384
_EMB = 64
_HID = 32

_NC = 2
_NS = 16
_NW = _NC * _NS
_BPW = _BATCH // _NW


@functools.cache
def _make_sc_gather():
    @functools.partial(
        pl.kernel,
        mesh=plsc.VectorSubcoreMesh(
            core_axis_name="c", subcore_axis_name="s",
            num_cores=_NC, num_subcores=_NS,
        ),
        out_type=jax.ShapeDtypeStruct((_BATCH, 2 * _EMB), jnp.float32),
        scratch_types=[
            pltpu.VMEM((_BPW,), jnp.int32),
            pltpu.VMEM((_BPW, _EMB), jnp.float32),
            pltpu.VMEM((_BPW,), jnp.int32),
            pltpu.VMEM((_BPW, _EMB), jnp.float32),
            pltpu.SemaphoreType.DMA,
            pltpu.SemaphoreType.DMA,
        ],
        compiler_params=pltpu.CompilerParams(use_tc_tiling_on_sc=False),
    )
    def sc_gather(user_hbm, quiz_hbm, ut_hbm, qt_hbm, x_out,
                  uidx_v, urows_v, qidx_v, qrows_v, sem_u, sem_q):
        wid = lax.axis_index("s") * _NC + lax.axis_index("c")
        base = wid * _BPW
        pltpu.sync_copy(user_hbm.at[pl.ds(base, _BPW)], uidx_v)
        pltpu.sync_copy(quiz_hbm.at[pl.ds(base, _BPW)], qidx_v)
        cu = pltpu.async_copy(ut_hbm.at[uidx_v], urows_v, sem_u)
        cq = pltpu.async_copy(qt_hbm.at[qidx_v], qrows_v, sem_q)
        cu.wait()
        pltpu.sync_copy(urows_v, x_out.at[pl.ds(base, _BPW), pl.ds(0, _EMB)])
        cq.wait()
        pltpu.sync_copy(qrows_v, x_out.at[pl.ds(base, _BPW), pl.ds(_EMB, _EMB)])

    return sc_gather


def _mlp_body(x_ref, t_ref, w1_ref, w1t_ref, b1_ref, w2_ref, b2_ref, out_ref):
    h = (
        jnp.dot(x_ref[...], w1_ref[...], preferred_element_type=jnp.float32)
        + t_ref[...] * w1t_ref[...]
        + b1_ref[...]
    )
    h = jnp.maximum(h, 0.0)
    o = jnp.dot(h, w2_ref[...], preferred_element_type=jnp.float32) + b2_ref[...]
    out_ref[...] = 1.0 / (1.0 + jnp.exp(-o))


_MLP_BLK = 2048


def _mlp(x, time, W1x, W1t, b1, W2, b2):
    grid = (_BATCH // _MLP_BLK,)
    full = lambda shape: pl.BlockSpec(shape, lambda i: (0, 0))
    return pl.pallas_call(
        _mlp_body,
        grid=grid,
        in_specs=[
            pl.BlockSpec((_MLP_BLK, 2 * _EMB), lambda i: (i, 0)),
            pl.BlockSpec((_MLP_BLK, 1), lambda i: (i, 0)),
            full((2 * _EMB, _HID)),
            full((1, _HID)),
            full((1, _HID)),
            full((_HID, 1)),
            full((1, 1)),
        ],
        out_specs=pl.BlockSpec((_MLP_BLK, 1), lambda i: (i, 0)),
        out_shape=jax.ShapeDtypeStruct((_BATCH, 1), jnp.float32),
    )(x, time, W1x, W1t, b1, W2, b2)


def kernel(user, quiz, time, user_table, quiz_table, W1, b1, W2, b2):
    x = _make_sc_gather()(user, quiz, user_table, quiz_table)
    W1x = W1[:2 * _EMB]
    W1t = W1[2 * _EMB:]
    out = _mlp(x, time, W1x, W1t, b1.reshape(1, _HID), W2, b2.reshape(1, 1))
    return out[:, 0]


# speedup vs baseline: 1.2800x; 1.0010x over previous
import functools

import jax
import jax.numpy as jnp
from jax import lax
from jax.experimental import pallas as pl
from jax.experimental.pallas import tpu as pltpu
from jax.experimental.pallas import tpu_sc as plsc

_BATCH = 16384
_EMB = 64
_HID = 32

_NC = 2
_NS = 16
_NW = _NC * _NS
_BPW = _BATCH // _NW


@functools.cache
def _make_sc_gather():
    @functools.partial(
        pl.kernel,
        mesh=plsc.VectorSubcoreMesh(
            core_axis_name="c", subcore_axis_name="s",
            num_cores=_NC, num_subcores=_NS,
        ),
        out_type=jax.ShapeDtypeStruct((_BATCH, 2 * _EMB), jnp.float32),
        scratch_types=[
            pltpu.VMEM((_BPW,), jnp.int32),
            pltpu.VMEM((_BPW, _EMB), jnp.float32),
            pltpu.VMEM((_BPW,), jnp.int32),
            pltpu.VMEM((_BPW, _EMB), jnp.float32),
            pltpu.SemaphoreType.DMA,
            pltpu.SemaphoreType.DMA,
        ],
        compiler_params=pltpu.CompilerParams(use_tc_tiling_on_sc=False),
    )
    def sc_gather(user_hbm, quiz_hbm, ut_hbm, qt_hbm, x_out,
                  uidx_v, urows_v, qidx_v, qrows_v, sem_u, sem_q):
        wid = lax.axis_index("s") * _NC + lax.axis_index("c")
        base = wid * _BPW
        pltpu.sync_copy(user_hbm.at[pl.ds(base, _BPW)], uidx_v)
        pltpu.sync_copy(quiz_hbm.at[pl.ds(base, _BPW)], qidx_v)
        cu = pltpu.async_copy(ut_hbm.at[uidx_v], urows_v, sem_u)
        cq = pltpu.async_copy(qt_hbm.at[qidx_v], qrows_v, sem_q)
        cu.wait()
        pltpu.sync_copy(urows_v, x_out.at[pl.ds(base, _BPW), pl.ds(0, _EMB)])
        cq.wait()
        pltpu.sync_copy(qrows_v, x_out.at[pl.ds(base, _BPW), pl.ds(_EMB, _EMB)])

    return sc_gather


def _mlp_body(x_ref, t_ref, w1_ref, w1t_ref, b1_ref, w2_ref, b2_ref, out_ref):
    h = (
        jnp.dot(x_ref[...], w1_ref[...], preferred_element_type=jnp.float32)
        + t_ref[...] * w1t_ref[...]
        + b1_ref[...]
    )
    h = jnp.maximum(h, 0.0)
    o = jnp.dot(h, w2_ref[...], preferred_element_type=jnp.float32) + b2_ref[...]
    out_ref[...] = 1.0 / (1.0 + jnp.exp(-o))


_MLP_BLK = 2048


def _mlp(x, time, W1x, W1t, b1, W2, b2):
    grid = (_BATCH // _MLP_BLK,)
    full = lambda shape: pl.BlockSpec(shape, lambda i: (0, 0))
    return pl.pallas_call(
        _mlp_body,
        grid=grid,
        in_specs=[
            pl.BlockSpec((_MLP_BLK, 2 * _EMB), lambda i: (i, 0)),
            pl.BlockSpec((_MLP_BLK, 1), lambda i: (i, 0)),
            full((2 * _EMB, _HID)),
            full((1, _HID)),
            full((1, _HID)),
            full((_HID, 1)),
            full((1, 1)),
        ],
        out_specs=pl.BlockSpec((_MLP_BLK, 1), lambda i: (i, 0)),
        out_shape=jax.ShapeDtypeStruct((_BATCH, 1), jnp.float32),
    )(x, time, W1x, W1t, b1, W2, b2)


def _linearize(table):
    flat = lax.optimization_barrier(table.reshape(-1))
    return flat.reshape(table.shape)


def kernel(user, quiz, time, user_table, quiz_table, W1, b1, W2, b2):
    x = _make_sc_gather()(user, quiz, _linearize(user_table),
                          _linearize(quiz_table))
    W1x = W1[:2 * _EMB]
    W1t = W1[2 * _EMB:]
    out = _mlp(x, time, W1x, W1t, b1.reshape(1, _HID), W2, b2.reshape(1, 1))
    return out[:, 0]


# speedup vs baseline: 1.3920x; 1.0876x over previous
import functools

import jax
import jax.numpy as jnp
from jax import lax
from jax.experimental import pallas as pl
from jax.experimental.pallas import tpu as pltpu
from jax.experimental.pallas import tpu_sc as plsc

_BATCH = 16384
_EMB = 64
_HID = 32

_NC = 2
_NS = 16
_NW = _NC * _NS
_BPW = _BATCH // _NW
_CHUNK = _BPW // 2


@functools.cache
def _make_sc_gather():
    @functools.partial(
        pl.kernel,
        mesh=plsc.VectorSubcoreMesh(
            core_axis_name="c", subcore_axis_name="s",
            num_cores=_NC, num_subcores=_NS,
        ),
        out_type=jax.ShapeDtypeStruct((_BATCH, 2 * _EMB), jnp.float32),
        scratch_types=[
            pltpu.VMEM((_BPW,), jnp.int32),
            pltpu.VMEM((_CHUNK, 2 * _EMB), jnp.float32),
            pltpu.VMEM((_BPW,), jnp.int32),
            pltpu.VMEM((_CHUNK, 2 * _EMB), jnp.float32),
            pltpu.SemaphoreType.DMA,
            pltpu.SemaphoreType.DMA,
        ],
        compiler_params=pltpu.CompilerParams(use_tc_tiling_on_sc=False),
    )
    def sc_gather(user_hbm, quiz_hbm, xt_hbm, x_out,
                  uidx_v, urows_v, qidx_v, qrows_v, sem_u, sem_q):
        wid = lax.axis_index("s") * _NC + lax.axis_index("c")
        base = wid * _BPW
        pltpu.sync_copy(user_hbm.at[pl.ds(base, _BPW)], uidx_v)
        pltpu.sync_copy(quiz_hbm.at[pl.ds(base, _BPW)], qidx_v)
        for k in range(_BPW // _CHUNK):
            off = k * _CHUNK
            cu = pltpu.async_copy(
                xt_hbm.at[uidx_v.at[pl.ds(off, _CHUNK)]], urows_v, sem_u)
            cq = pltpu.async_copy(
                xt_hbm.at[qidx_v.at[pl.ds(off, _CHUNK)]], qrows_v, sem_q)
            cu.wait()
            pltpu.sync_copy(urows_v.at[:, pl.ds(0, _EMB)],
                            x_out.at[pl.ds(base + off, _CHUNK), pl.ds(0, _EMB)])
            cq.wait()
            pltpu.sync_copy(qrows_v.at[:, pl.ds(_EMB, _EMB)],
                            x_out.at[pl.ds(base + off, _CHUNK),
                                     pl.ds(_EMB, _EMB)])

    return sc_gather


def _mlp_body(x_ref, t_ref, w1_ref, w1t_ref, b1_ref, w2_ref, b2_ref, out_ref):
    h = (
        jnp.dot(x_ref[...], w1_ref[...], preferred_element_type=jnp.float32)
        + t_ref[...] * w1t_ref[...]
        + b1_ref[...]
    )
    h = jnp.maximum(h, 0.0)
    o = jnp.dot(h, w2_ref[...], preferred_element_type=jnp.float32) + b2_ref[...]
    out_ref[...] = 1.0 / (1.0 + jnp.exp(-o))


_MLP_BLK = 2048


def _mlp(x, time, W1x, W1t, b1, W2, b2):
    grid = (_BATCH // _MLP_BLK,)
    full = lambda shape: pl.BlockSpec(shape, lambda i: (0, 0))
    return pl.pallas_call(
        _mlp_body,
        grid=grid,
        in_specs=[
            pl.BlockSpec((_MLP_BLK, 2 * _EMB), lambda i: (i, 0)),
            pl.BlockSpec((_MLP_BLK, 1), lambda i: (i, 0)),
            full((2 * _EMB, _HID)),
            full((1, _HID)),
            full((1, _HID)),
            full((_HID, 1)),
            full((1, 1)),
        ],
        out_specs=pl.BlockSpec((_MLP_BLK, 1), lambda i: (i, 0)),
        out_shape=jax.ShapeDtypeStruct((_BATCH, 1), jnp.float32),
    )(x, time, W1x, W1t, b1, W2, b2)


def kernel(user, quiz, time, user_table, quiz_table, W1, b1, W2, b2):
    xt = jnp.concatenate([user_table, quiz_table], axis=1)
    x = _make_sc_gather()(user, quiz, xt)
    W1x = W1[:2 * _EMB]
    W1t = W1[2 * _EMB:]
    out = _mlp(x, time, W1x, W1t, b1.reshape(1, _HID), W2, b2.reshape(1, 1))
    return out[:, 0]


# speedup vs baseline: 1.7272x; 1.2408x over previous
import functools

import jax
import jax.numpy as jnp
from jax import lax
from jax.experimental import pallas as pl
from jax.experimental.pallas import tpu as pltpu
from jax.experimental.pallas import tpu_sc as plsc

_BATCH = 16384
_EMB = 64
_HID = 32

_NC = 2
_NS = 16
_NW = _NC * _NS
_BPW = _BATCH // _NW
_CHUNK = _BPW // 2


@functools.cache
def _make_sc_gather():
    @functools.partial(
        pl.kernel,
        mesh=plsc.VectorSubcoreMesh(
            core_axis_name="c", subcore_axis_name="s",
            num_cores=_NC, num_subcores=_NS,
        ),
        out_type=jax.ShapeDtypeStruct((_BATCH, 2 * _EMB), jnp.float32),
        scratch_types=[
            pltpu.VMEM((_BPW,), jnp.int32),
            pltpu.VMEM((_CHUNK, 2 * _EMB), jnp.float32),
            pltpu.VMEM((_BPW,), jnp.int32),
            pltpu.VMEM((_CHUNK, 2 * _EMB), jnp.float32),
            pltpu.SemaphoreType.DMA,
            pltpu.SemaphoreType.DMA,
        ],
        compiler_params=pltpu.CompilerParams(use_tc_tiling_on_sc=False),
    )
    def sc_gather(user_hbm, quiz_hbm, xt_hbm, x_out,
                  uidx_v, urows_v, qidx_v, qrows_v, sem_u, sem_q):
        wid = lax.axis_index("s") * _NC + lax.axis_index("c")
        base = wid * _BPW
        pltpu.sync_copy(user_hbm.at[pl.ds(base, _BPW)], uidx_v)
        pltpu.sync_copy(quiz_hbm.at[pl.ds(base, _BPW)], qidx_v)
        for k in range(_BPW // _CHUNK):
            off = k * _CHUNK
            cu = pltpu.async_copy(
                xt_hbm.at[uidx_v.at[pl.ds(off, _CHUNK)]], urows_v, sem_u)
            cq = pltpu.async_copy(
                xt_hbm.at[qidx_v.at[pl.ds(off, _CHUNK)]], qrows_v, sem_q)
            cu.wait()
            pltpu.sync_copy(urows_v.at[:, pl.ds(0, _EMB)],
                            x_out.at[pl.ds(base + off, _CHUNK), pl.ds(0, _EMB)])
            cq.wait()
            pltpu.sync_copy(qrows_v.at[:, pl.ds(_EMB, _EMB)],
                            x_out.at[pl.ds(base + off, _CHUNK),
                                     pl.ds(_EMB, _EMB)])

    return sc_gather


def _merge_body(u_ref, q_ref, out_ref):
    out_ref[...] = jnp.concatenate(
        [u_ref[...].T, q_ref[...].T], axis=1)


_MERGE_BLK = 2048


def _merge(uT, qT):
    n = uT.shape[1]
    grid = (pl.cdiv(n, _MERGE_BLK),)
    return pl.pallas_call(
        _merge_body,
        grid=grid,
        in_specs=[
            pl.BlockSpec((_EMB, _MERGE_BLK), lambda i: (0, i)),
            pl.BlockSpec((_EMB, _MERGE_BLK), lambda i: (0, i)),
        ],
        out_specs=pl.BlockSpec((_MERGE_BLK, 2 * _EMB), lambda i: (i, 0)),
        out_shape=jax.ShapeDtypeStruct((n, 2 * _EMB), jnp.float32),
    )(uT, qT)


def _mlp_body(x_ref, t_ref, w1_ref, w1t_ref, b1_ref, w2_ref, b2_ref, out_ref):
    h = (
        jnp.dot(x_ref[...], w1_ref[...], preferred_element_type=jnp.float32)
        + t_ref[...] * w1t_ref[...]
        + b1_ref[...]
    )
    h = jnp.maximum(h, 0.0)
    o = jnp.dot(h, w2_ref[...], preferred_element_type=jnp.float32) + b2_ref[...]
    out_ref[...] = 1.0 / (1.0 + jnp.exp(-o))


_MLP_BLK = 2048


def _mlp(x, time, W1x, W1t, b1, W2, b2):
    grid = (_BATCH // _MLP_BLK,)
    full = lambda shape: pl.BlockSpec(shape, lambda i: (0, 0))
    return pl.pallas_call(
        _mlp_body,
        grid=grid,
        in_specs=[
            pl.BlockSpec((_MLP_BLK, 2 * _EMB), lambda i: (i, 0)),
            pl.BlockSpec((_MLP_BLK, 1), lambda i: (i, 0)),
            full((2 * _EMB, _HID)),
            full((1, _HID)),
            full((1, _HID)),
            full((_HID, 1)),
            full((1, 1)),
        ],
        out_specs=pl.BlockSpec((_MLP_BLK, 1), lambda i: (i, 0)),
        out_shape=jax.ShapeDtypeStruct((_BATCH, 1), jnp.float32),
    )(x, time, W1x, W1t, b1, W2, b2)


def kernel(user, quiz, time, user_table, quiz_table, W1, b1, W2, b2):
    xt = _merge(user_table.T, quiz_table.T)
    x = _make_sc_gather()(user, quiz, xt)
    W1x = W1[:2 * _EMB]
    W1t = W1[2 * _EMB:]
    out = _mlp(x, time, W1x, W1t, b1.reshape(1, _HID), W2, b2.reshape(1, 1))
    return out[:, 0]


# speedup vs baseline: 2.0420x; 1.1823x over previous
import functools

import jax
import jax.numpy as jnp
from jax import lax
from jax.experimental import pallas as pl
from jax.experimental.pallas import tpu as pltpu
from jax.experimental.pallas import tpu_sc as plsc

_BATCH = 16384
_EMB = 64
_HID = 32

_NC = 2
_NS = 16
_NW = _NC * _NS
_BPW = _BATCH // _NW
_CHUNK = _BPW // 2


@functools.cache
def _make_sc_gather():
    @functools.partial(
        pl.kernel,
        mesh=plsc.VectorSubcoreMesh(
            core_axis_name="c", subcore_axis_name="s",
            num_cores=_NC, num_subcores=_NS,
        ),
        out_type=jax.ShapeDtypeStruct((_BATCH, 2 * _EMB), jnp.float32),
        scratch_types=[
            pltpu.VMEM((_BPW,), jnp.int32),
            pltpu.VMEM((_CHUNK, 2 * _EMB), jnp.float32),
            pltpu.VMEM((_BPW,), jnp.int32),
            pltpu.VMEM((_CHUNK, 2 * _EMB), jnp.float32),
            pltpu.SemaphoreType.DMA,
            pltpu.SemaphoreType.DMA,
        ],
        compiler_params=pltpu.CompilerParams(use_tc_tiling_on_sc=False),
    )
    def sc_gather(user_hbm, quiz_hbm, xt_hbm, x_out,
                  uidx_v, urows_v, qidx_v, qrows_v, sem_u, sem_q):
        wid = lax.axis_index("s") * _NC + lax.axis_index("c")
        base = wid * _BPW
        pltpu.sync_copy(user_hbm.at[pl.ds(base, _BPW)], uidx_v)
        pltpu.sync_copy(quiz_hbm.at[pl.ds(base, _BPW)], qidx_v)
        for k in range(_BPW // _CHUNK):
            off = k * _CHUNK
            cu = pltpu.async_copy(
                xt_hbm.at[uidx_v.at[pl.ds(off, _CHUNK)]], urows_v, sem_u)
            cq = pltpu.async_copy(
                xt_hbm.at[qidx_v.at[pl.ds(off, _CHUNK)]], qrows_v, sem_q)
            cu.wait()
            pltpu.sync_copy(urows_v.at[:, pl.ds(0, _EMB)],
                            x_out.at[pl.ds(base + off, _CHUNK), pl.ds(0, _EMB)])
            cq.wait()
            pltpu.sync_copy(qrows_v.at[:, pl.ds(_EMB, _EMB)],
                            x_out.at[pl.ds(base + off, _CHUNK),
                                     pl.ds(_EMB, _EMB)])

    return sc_gather


def _merge_body(u_ref, q_ref, out_ref):
    out_ref[...] = jnp.concatenate(
        [u_ref[...].T, q_ref[...].T], axis=1)


_MERGE_BLK = 8192


def _merge(uT, qT):
    n = uT.shape[1]
    grid = (pl.cdiv(n, _MERGE_BLK),)
    return pl.pallas_call(
        _merge_body,
        grid=grid,
        in_specs=[
            pl.BlockSpec((_EMB, _MERGE_BLK), lambda i: (0, i)),
            pl.BlockSpec((_EMB, _MERGE_BLK), lambda i: (0, i)),
        ],
        out_specs=pl.BlockSpec((_MERGE_BLK, 2 * _EMB), lambda i: (i, 0)),
        out_shape=jax.ShapeDtypeStruct((n, 2 * _EMB), jnp.float32),
    )(uT, qT)


def _mlp_body(x_ref, t_ref, w1_ref, w1t_ref, b1_ref, w2_ref, b2_ref, out_ref):
    h = (
        jnp.dot(x_ref[...], w1_ref[...], preferred_element_type=jnp.float32)
        + t_ref[...] * w1t_ref[...]
        + b1_ref[...]
    )
    h = jnp.maximum(h, 0.0)
    o = jnp.dot(h, w2_ref[...], preferred_element_type=jnp.float32) + b2_ref[...]
    out_ref[...] = 1.0 / (1.0 + jnp.exp(-o))


_MLP_BLK = 2048


def _mlp(x, time, W1x, W1t, b1, W2, b2):
    grid = (_BATCH // _MLP_BLK,)
    full = lambda shape: pl.BlockSpec(shape, lambda i: (0, 0))
    return pl.pallas_call(
        _mlp_body,
        grid=grid,
        in_specs=[
            pl.BlockSpec((_MLP_BLK, 2 * _EMB), lambda i: (i, 0)),
            pl.BlockSpec((_MLP_BLK, 1), lambda i: (i, 0)),
            full((2 * _EMB, _HID)),
            full((1, _HID)),
            full((1, _HID)),
            full((_HID, 1)),
            full((1, 1)),
        ],
        out_specs=pl.BlockSpec((_MLP_BLK, 1), lambda i: (i, 0)),
        out_shape=jax.ShapeDtypeStruct((_BATCH, 1), jnp.float32),
    )(x, time, W1x, W1t, b1, W2, b2)


def kernel(user, quiz, time, user_table, quiz_table, W1, b1, W2, b2):
    xt = _merge(user_table.T, quiz_table.T)
    x = _make_sc_gather()(user, quiz, xt)
    W1x = W1[:2 * _EMB]
    W1t = W1[2 * _EMB:]
    out = _mlp(x, time, W1x, W1t, b1.reshape(1, _HID), W2, b2.reshape(1, 1))
    return out[:, 0]
